# baseline (speedup 1.0000x reference)
import numpy as np
import jax
import jax.numpy as jnp
from jax import lax
from jax.experimental import pallas as pl
from jax.experimental.pallas import tpu as pltpu
from jax.experimental.pallas import tpu_sc as plsc

_VOCAB = 100000
_D = 1024
_B = 4
_S = 8192
_N = _B * _S
_NC, _NS = 2, 16
_NW = _NC * _NS
_PPW = _S // _NW
_C = 16
_CHUNKS = _PPW // _C
_NPAIR = _CHUNKS // 2
_LANES = 16


def _make_pe():
    pos = np.arange(_S, dtype=np.float32)[:, None]
    i = np.arange(0, _D, 2, dtype=np.float32)
    div = np.exp(-(np.log(10000.0)) * i / np.float32(_D)).astype(np.float32)
    ang = pos * div[None, :]
    pe = np.zeros((_S, _D), dtype=np.float32)
    pe[:, 0::2] = np.sin(ang)
    pe[:, 1::2] = np.cos(ang)
    return pe


_PE = _make_pe()


def _body(x_hbm, pe_hbm, tab_hbm, out_hbm, idx_all, pe_v, rows, gsem0, gsem1,
          ssem0, ssem1, pesem):
    c = lax.axis_index("c")
    s = lax.axis_index("s")
    wid = s * _NC + c
    p0 = wid * _PPW
    gsem = (gsem0, gsem1)
    ssem = (ssem0, ssem1)

    for b in range(_B):
        pltpu.sync_copy(x_hbm.at[pl.ds(b * _S + p0, _PPW)], idx_all.at[b])

    pltpu.async_copy(pe_hbm.at[pl.ds(p0, _C)], pe_v.at[0], pesem)
    pltpu.async_copy(tab_hbm.at[idx_all.at[0, pl.ds(0, _C)]], rows.at[0],
                     gsem0)

    def pair_body(i, carry):
        st_desc = [None, None]
        g_desc = [None, None]
        for half in (0, 1):
            k = 2 * i + half
            s0 = p0 + k * _C
            pltpu.make_async_copy(
                pe_hbm.at[pl.ds(0, _C)], pe_v.at[half], pesem).wait()
            if half == 0:
                pltpu.async_copy(pe_hbm.at[pl.ds(s0 + _C, _C)],
                                 pe_v.at[1], pesem)
            else:
                @pl.when(i < _NPAIR - 1)
                def _():
                    pltpu.async_copy(pe_hbm.at[pl.ds(s0 + _C, _C)],
                                     pe_v.at[0], pesem)
            for b in range(_B):
                j = 4 * half + b
                q = j % 2
                if j == 0:
                    pltpu.make_async_copy(
                        tab_hbm.at[idx_all.at[0, pl.ds(0, _C)]], rows.at[0],
                        gsem0).wait()
                else:
                    g_desc[q].wait()
                if j == 0:
                    @pl.when(i > 0)
                    def _():
                        pltpu.make_async_copy(
                            rows.at[1], out_hbm.at[pl.ds(0, _C)],
                            ssem1).wait()
                else:
                    st_desc[1 - q].wait()
                if j < 7:
                    kn = 2 * i + (j + 1) // 4
                    bn = (j + 1) % 4
                    g_desc[1 - q] = pltpu.async_copy(
                        tab_hbm.at[idx_all.at[bn, pl.ds(kn * _C, _C)]],
                        rows.at[1 - q], gsem[1 - q])
                else:
                    @pl.when(i < _NPAIR - 1)
                    def _():
                        pltpu.async_copy(
                            tab_hbm.at[idx_all.at[0, pl.ds((k + 1) * _C, _C)]],
                            rows.at[0], gsem0)
                def add_row(r, carry2):
                    for v in range(_D // _LANES):
                        sl = pl.ds(v * _LANES, _LANES)
                        plsc.addupdate(rows.at[q, r, sl], pe_v[half, r, sl])
                    return carry2
                lax.fori_loop(0, _C, add_row, 0)
                st_desc[q] = pltpu.async_copy(
                    rows.at[q], out_hbm.at[pl.ds(b * _S + s0, _C)], ssem[q])
        return carry

    lax.fori_loop(0, _NPAIR, pair_body, 0)

    pltpu.make_async_copy(rows.at[1], out_hbm.at[pl.ds(0, _C)], ssem1).wait()


def kernel(x, tok_table):
    x_flat = x.reshape(_N)
    mesh = plsc.VectorSubcoreMesh(
        core_axis_name="c", subcore_axis_name="s",
        num_cores=_NC, num_subcores=_NS)
    f = pl.kernel(
        _body,
        out_type=jax.ShapeDtypeStruct((_N, _D), jnp.float32),
        mesh=mesh,
        scratch_types=[
            pltpu.VMEM((_B, _PPW), jnp.int32),
            pltpu.VMEM((2, _C, _D), jnp.float32),
            pltpu.VMEM((2, _C, _D), jnp.float32),
            pltpu.SemaphoreType.DMA,
            pltpu.SemaphoreType.DMA,
            pltpu.SemaphoreType.DMA,
            pltpu.SemaphoreType.DMA,
            pltpu.SemaphoreType.DMA,
        ],
    )
    out = f(x_flat, _PE, tok_table)
    return out.reshape(_B, _S, _D)


# speedup vs baseline: 1.1873x; 1.1873x over previous
import numpy as np
import jax
import jax.numpy as jnp
from jax import lax
from jax.experimental import pallas as pl
from jax.experimental.pallas import tpu as pltpu
from jax.experimental.pallas import tpu_sc as plsc

_VOCAB = 100000
_D = 1024
_B = 4
_S = 8192
_N = _B * _S
_NC, _NS = 2, 16
_NW = _NC * _NS
_PPW = _S // _NW
_C = 16
_CHUNKS = _PPW // _C
_NPAIR = _CHUNKS // 2
_LANES = 16


def _make_pe():
    pos = np.arange(_S, dtype=np.float32)[:, None]
    i = np.arange(0, _D, 2, dtype=np.float32)
    div = np.exp(-(np.log(10000.0)) * i / np.float32(_D)).astype(np.float32)
    ang = pos * div[None, :]
    pe = np.zeros((_S, _D), dtype=np.float32)
    pe[:, 0::2] = np.sin(ang)
    pe[:, 1::2] = np.cos(ang)
    return pe


_PE = _make_pe()


def _body(x_hbm, pe_hbm, tab_hbm, out_hbm, idx_all, pe_v, rows, gsem0, gsem1,
          ssem0, ssem1, pesem):
    c = lax.axis_index("c")
    s = lax.axis_index("s")
    wid = s * _NC + c
    p0 = wid * _PPW
    gsem = (gsem0, gsem1)
    ssem = (ssem0, ssem1)

    for b in range(_B):
        pltpu.sync_copy(x_hbm.at[pl.ds(b * _S + p0, _PPW)], idx_all.at[b])

    pltpu.async_copy(pe_hbm.at[pl.ds(p0, _C)], pe_v.at[0], pesem)
    pltpu.async_copy(tab_hbm.at[idx_all.at[0, pl.ds(0, _C)]], rows.at[0],
                     gsem0)

    def pair_body(i, carry):
        st_desc = [None, None]
        g_desc = [None, None]
        for half in (0, 1):
            k = 2 * i + half
            s0 = p0 + k * _C
            pltpu.make_async_copy(
                pe_hbm.at[pl.ds(0, _C)], pe_v.at[half], pesem).wait()
            if half == 0:
                pltpu.async_copy(pe_hbm.at[pl.ds(s0 + _C, _C)],
                                 pe_v.at[1], pesem)
            else:
                @pl.when(i < _NPAIR - 1)
                def _():
                    pltpu.async_copy(pe_hbm.at[pl.ds(s0 + _C, _C)],
                                     pe_v.at[0], pesem)
            for b in range(_B):
                j = 4 * half + b
                q = j % 2
                if j == 0:
                    pltpu.make_async_copy(
                        tab_hbm.at[idx_all.at[0, pl.ds(0, _C)]], rows.at[0],
                        gsem0).wait()
                else:
                    g_desc[q].wait()
                if j == 0:
                    @pl.when(i > 0)
                    def _():
                        pltpu.make_async_copy(
                            rows.at[1], out_hbm.at[pl.ds(0, _C)],
                            ssem1).wait()
                else:
                    st_desc[1 - q].wait()
                if j < 7:
                    kn = 2 * i + (j + 1) // 4
                    bn = (j + 1) % 4
                    g_desc[1 - q] = pltpu.async_copy(
                        tab_hbm.at[idx_all.at[bn, pl.ds(kn * _C, _C)]],
                        rows.at[1 - q], gsem[1 - q])
                else:
                    @pl.when(i < _NPAIR - 1)
                    def _():
                        pltpu.async_copy(
                            tab_hbm.at[idx_all.at[0, pl.ds((k + 1) * _C, _C)]],
                            rows.at[0], gsem0)
                def add_row(r, carry2):
                    for v in range(_D // _LANES):
                        sl = pl.ds(v * _LANES, _LANES)
                        rows[q, r, sl] = rows[q, r, sl] + pe_v[half, r, sl]
                    return carry2
                lax.fori_loop(0, _C, add_row, 0)
                st_desc[q] = pltpu.async_copy(
                    rows.at[q], out_hbm.at[pl.ds(b * _S + s0, _C)], ssem[q])
        return carry

    lax.fori_loop(0, _NPAIR, pair_body, 0)

    pltpu.make_async_copy(rows.at[1], out_hbm.at[pl.ds(0, _C)], ssem1).wait()


def kernel(x, tok_table):
    x_flat = x.reshape(_N)
    mesh = plsc.VectorSubcoreMesh(
        core_axis_name="c", subcore_axis_name="s",
        num_cores=_NC, num_subcores=_NS)
    f = pl.kernel(
        _body,
        out_type=jax.ShapeDtypeStruct((_N, _D), jnp.float32),
        mesh=mesh,
        scratch_types=[
            pltpu.VMEM((_B, _PPW), jnp.int32),
            pltpu.VMEM((2, _C, _D), jnp.float32),
            pltpu.VMEM((2, _C, _D), jnp.float32),
            pltpu.SemaphoreType.DMA,
            pltpu.SemaphoreType.DMA,
            pltpu.SemaphoreType.DMA,
            pltpu.SemaphoreType.DMA,
            pltpu.SemaphoreType.DMA,
        ],
    )
    out = f(x_flat, _PE, tok_table)
    return out.reshape(_B, _S, _D)


# speedup vs baseline: 2.1337x; 1.7971x over previous
import numpy as np
import jax
import jax.numpy as jnp
from jax import lax
from jax.experimental import pallas as pl
from jax.experimental.pallas import tpu as pltpu
from jax.experimental.pallas import tpu_sc as plsc

_VOCAB = 100000
_D = 1024
_B = 4
_S = 8192
_N = _B * _S
_NC, _NS = 2, 16
_NW = _NC * _NS
_PPW = _S // _NW
_C = 8
_CHUNKS = _PPW // _C
_NPAIR = _CHUNKS // 2
_LANES = 16


def _make_pe():
    pos = np.arange(_S, dtype=np.float32)[:, None]
    i = np.arange(0, _D, 2, dtype=np.float32)
    div = np.exp(-(np.log(10000.0)) * i / np.float32(_D)).astype(np.float32)
    ang = pos * div[None, :]
    pe = np.zeros((_S, _D), dtype=np.float32)
    pe[:, 0::2] = np.sin(ang)
    pe[:, 1::2] = np.cos(ang)
    return pe


_PE = _make_pe()


def _body(x_hbm, pe_hbm, tab_hbm, out_hbm, idx_all, pe_v, rows, gsem0, gsem1,
          ssem0, ssem1, pesem):
    c = lax.axis_index("c")
    s = lax.axis_index("s")
    wid = s * _NC + c
    p0 = wid * _PPW
    gsem = (gsem0, gsem1)
    ssem = (ssem0, ssem1)

    def launch_gathers(k, p):
        for b in range(_B):
            pltpu.async_copy(
                tab_hbm.at[idx_all.at[b, pl.ds(k * _C, _C)]],
                rows.at[p, b], gsem[p])

    def drain(n, src, dst, sem):
        for _ in range(n):
            pltpu.make_async_copy(src, dst, sem).wait()

    for b in range(_B):
        pltpu.sync_copy(x_hbm.at[pl.ds(b * _S + p0, _PPW)], idx_all.at[b])

    pltpu.async_copy(pe_hbm.at[pl.ds(p0, _C)], pe_v.at[0], pesem)
    launch_gathers(0, 0)

    def pair_body(i, carry):
        for half in (0, 1):
            k = 2 * i + half
            s0 = p0 + k * _C
            pltpu.make_async_copy(
                pe_hbm.at[pl.ds(0, _C)], pe_v.at[half], pesem).wait()
            if half == 0:
                pltpu.async_copy(pe_hbm.at[pl.ds(s0 + _C, _C)],
                                 pe_v.at[1], pesem)
            else:
                @pl.when(i < _NPAIR - 1)
                def _():
                    pltpu.async_copy(pe_hbm.at[pl.ds(s0 + _C, _C)],
                                     pe_v.at[0], pesem)
            drain(_B, tab_hbm.at[idx_all.at[0, pl.ds(0, _C)]],
                  rows.at[half, 0], gsem[half])
            if half == 0:
                @pl.when(i > 0)
                def _():
                    drain(_B, rows.at[1, 0], out_hbm.at[pl.ds(0, _C)], ssem1)
            else:
                drain(_B, rows.at[0, 0], out_hbm.at[pl.ds(0, _C)], ssem0)
            if half == 0:
                launch_gathers(k + 1, 1)
            else:
                @pl.when(i < _NPAIR - 1)
                def _():
                    launch_gathers(k + 1, 0)

            def add_row(r, carry2):
                for v in range(_D // _LANES):
                    sl = pl.ds(v * _LANES, _LANES)
                    pe = pe_v[half, r, sl]
                    for b in range(_B):
                        rows[half, b, r, sl] = rows[half, b, r, sl] + pe
                return carry2
            lax.fori_loop(0, _C, add_row, 0)

            for b in range(_B):
                pltpu.async_copy(rows.at[half, b],
                                 out_hbm.at[pl.ds(b * _S + s0, _C)],
                                 ssem[half])
        return carry

    lax.fori_loop(0, _NPAIR, pair_body, 0)

    drain(_B, rows.at[1, 0], out_hbm.at[pl.ds(0, _C)], ssem1)


def kernel(x, tok_table):
    x_flat = x.reshape(_N)
    mesh = plsc.VectorSubcoreMesh(
        core_axis_name="c", subcore_axis_name="s",
        num_cores=_NC, num_subcores=_NS)
    f = pl.kernel(
        _body,
        out_type=jax.ShapeDtypeStruct((_N, _D), jnp.float32),
        mesh=mesh,
        scratch_types=[
            pltpu.VMEM((_B, _PPW), jnp.int32),
            pltpu.VMEM((2, _C, _D), jnp.float32),
            pltpu.VMEM((2, _B, _C, _D), jnp.float32),
            pltpu.SemaphoreType.DMA,
            pltpu.SemaphoreType.DMA,
            pltpu.SemaphoreType.DMA,
            pltpu.SemaphoreType.DMA,
            pltpu.SemaphoreType.DMA,
        ],
    )
    out = f(x_flat, _PE, tok_table)
    return out.reshape(_B, _S, _D)
